# baseline (speedup 1.0000x reference)
import functools

import jax
import jax.numpy as jnp
from jax import lax
from jax.experimental import pallas as pl
from jax.experimental.pallas import tpu as pltpu
from jax.experimental.pallas import tpu_sc as plsc

NC = 2
NS = 16
NW = NC * NS
L = 16
CHUNK = 128


def _sc_edge_aggregate(x, ei, n_pad, nq, nr, trash):
    N, D = x.shape
    rpt = n_pad // NS

    mesh = plsc.VectorSubcoreMesh(
        core_axis_name="c", subcore_axis_name="s",
        num_cores=NC, num_subcores=NS)

    def body(x_hbm, ei_hbm, aggr_hbm, deg_hbm,
             srcd0, srcd1, dstd0, dstd1, idxb0, idxb1, rb0, rb1,
             deg_v, aggr_sp, sem_i0, sem_i1, sem_g0, sem_g1, sem_s0, sem_s1):
        cid = lax.axis_index("c")
        sid = lax.axis_index("s")
        wid = cid * NS + sid
        nch = nq + jnp.where(wid < nr, 1, 0)
        base = (wid * nq + jnp.minimum(wid, nr)) * CHUNK

        zeros16 = jnp.zeros((L,), jnp.float32)
        ones16 = jnp.ones((L,), jnp.float32)
        trash16 = jnp.full((L,), trash, jnp.int32)

        def idx_dma_start(k, srcd_b, dstd_b, sem):
            off = base + k * CHUNK
            pltpu.async_copy(ei_hbm.at[0, pl.ds(off, CHUNK)], srcd_b, sem)
            pltpu.async_copy(ei_hbm.at[1, pl.ds(off, CHUNK)], dstd_b, sem)

        def idx_dma_wait(srcd_b, dstd_b, sem):
            pltpu.make_async_copy(ei_hbm.at[0, pl.ds(0, CHUNK)], srcd_b, sem).wait()
            pltpu.make_async_copy(ei_hbm.at[1, pl.ds(0, CHUNK)], dstd_b, sem).wait()

        idx_dma_start(0, srcd0, dstd0, sem_i0)
        idx_dma_start(1, srcd1, dstd1, sem_i1)

        idx_dma_wait(srcd0, dstd0, sem_i0)
        pltpu.async_copy(x_hbm.at[dstd0], rb0, sem_g0)

        def zrb(i, c):
            for cc in range(D // L):
                rb1[i, pl.ds(cc * L, L)] = zeros16
            return c
        lax.fori_loop(0, CHUNK, zrb, 0)

        def zdeg(i, c):
            deg_v[pl.ds(i * L, L)] = zeros16
            return c
        lax.fori_loop(0, n_pad // L, zdeg, 0)

        rowbase = sid * rpt
        nfull, rem = rpt // CHUNK, rpt % CHUNK
        for kk in range(nfull):
            pltpu.sync_copy(rb1, aggr_sp.at[pl.ds(rowbase + kk * CHUNK, CHUNK)])
        if rem:
            pltpu.sync_copy(rb1.at[pl.ds(0, rem)],
                            aggr_sp.at[pl.ds(rowbase + nfull * CHUNK, rem)])
        plsc.subcore_barrier()

        def step(j, srcd_b, dstd_b, idxb_b, rb_b, sem_g_b, sem_s_b,
                 srcd_o, dstd_o, idxb_o, rb_o, sem_g_o, sem_s_o,
                 sem_i_b, sem_i_o):
            for i in range(CHUNK // L):
                s16 = srcd_b[pl.ds(i * L, L)]
                d16 = dstd_b[pl.ds(i * L, L)]
                keep = s16 != d16
                idx16 = jnp.where(keep, s16, trash16)
                idxb_b[pl.ds(i * L, L)] = idx16
                plsc.addupdate_scatter(deg_v, [idx16], ones16)
            pltpu.make_async_copy(x_hbm.at[dstd_b], rb_b, sem_g_b).wait()

            @pl.when(j + 2 < nch)
            def _():
                idx_dma_start(j + 2, srcd_b, dstd_b, sem_i_b)

            @pl.when(j > 0)
            def _():
                pltpu.make_async_copy(rb_o, aggr_sp.at[idxb_o], sem_s_o).wait()

            @pl.when(j + 1 < nch)
            def _():
                idx_dma_wait(srcd_o, dstd_o, sem_i_o)
                pltpu.async_copy(x_hbm.at[dstd_o], rb_o, sem_g_o)

            pltpu.async_copy(rb_b, aggr_sp.at[idxb_b], sem_s_b, add=True)

        def loop(j, c):
            @pl.when(j % 2 == 0)
            def _():
                step(j, srcd0, dstd0, idxb0, rb0, sem_g0, sem_s0,
                     srcd1, dstd1, idxb1, rb1, sem_g1, sem_s1, sem_i0, sem_i1)

            @pl.when(j % 2 == 1)
            def _():
                step(j, srcd1, dstd1, idxb1, rb1, sem_g1, sem_s1,
                     srcd0, dstd0, idxb0, rb0, sem_g0, sem_s0, sem_i1, sem_i0)
            return c
        lax.fori_loop(0, nch, loop, 0)

        @pl.when((nch - 1) % 2 == 0)
        def _():
            pltpu.make_async_copy(rb0, aggr_sp.at[idxb0], sem_s0).wait()

        @pl.when((nch - 1) % 2 == 1)
        def _():
            pltpu.make_async_copy(rb1, aggr_sp.at[idxb1], sem_s1).wait()
        plsc.subcore_barrier()
        pltpu.sync_copy(aggr_sp.at[pl.ds(rowbase, rpt)],
                        aggr_hbm.at[cid, pl.ds(rowbase, rpt)])
        pltpu.sync_copy(deg_v, deg_hbm.at[wid])

    f = pl.kernel(
        body,
        out_type=[jax.ShapeDtypeStruct((NC, n_pad, D), jnp.float32),
                  jax.ShapeDtypeStruct((NW, n_pad), jnp.float32)],
        mesh=mesh,
        scratch_types=[
            pltpu.VMEM((CHUNK,), jnp.int32),
            pltpu.VMEM((CHUNK,), jnp.int32),
            pltpu.VMEM((CHUNK,), jnp.int32),
            pltpu.VMEM((CHUNK,), jnp.int32),
            pltpu.VMEM((CHUNK,), jnp.int32),
            pltpu.VMEM((CHUNK,), jnp.int32),
            pltpu.VMEM((CHUNK, D), jnp.float32),
            pltpu.VMEM((CHUNK, D), jnp.float32),
            pltpu.VMEM((n_pad,), jnp.float32),
            pltpu.VMEM_SHARED((n_pad, D), jnp.float32),
            pltpu.SemaphoreType.DMA,
            pltpu.SemaphoreType.DMA,
            pltpu.SemaphoreType.DMA,
            pltpu.SemaphoreType.DMA,
            pltpu.SemaphoreType.DMA,
            pltpu.SemaphoreType.DMA,
        ],
        compiler_params=pltpu.CompilerParams(needs_layout_passes=False),
    )
    return f(x, ei)


def _lin_body(x_ref, w1_ref, b1_ref, w2_ref, b2_ref, l1_ref, l2_ref):
    x = x_ref[...]
    l1_ref[...] = jnp.dot(x, w1_ref[...],
                          preferred_element_type=jnp.float32) + b1_ref[...]
    l2_ref[...] = jnp.dot(x, w2_ref[...],
                          preferred_element_type=jnp.float32) + b2_ref[...]


def _lin(x, W1, b1, W2, b2):
    N, D = x.shape
    BN = 2048
    nblk = (N + BN - 1) // BN
    return pl.pallas_call(
        _lin_body,
        grid=(nblk,),
        in_specs=[
            pl.BlockSpec((BN, D), lambda i: (i, 0)),
            pl.BlockSpec((D, D), lambda i: (0, 0)),
            pl.BlockSpec((1, D), lambda i: (0, 0)),
            pl.BlockSpec((D, D), lambda i: (0, 0)),
            pl.BlockSpec((1, D), lambda i: (0, 0)),
        ],
        out_specs=[pl.BlockSpec((BN, D), lambda i: (i, 0)),
                   pl.BlockSpec((BN, D), lambda i: (i, 0))],
        out_shape=[jax.ShapeDtypeStruct((N, D), jnp.float32),
                   jax.ShapeDtypeStruct((N, D), jnp.float32)],
    )(x, W1, b1, W2, b2)


def _combine_body(l1_ref, l2_ref, w_ref, ag_ref, dg_ref, o_ref):
    ax = ag_ref[0] + ag_ref[1]
    ag = jnp.dot(ax, w_ref[...], preferred_element_type=jnp.float32)
    deg = jnp.sum(dg_ref[...], axis=0)
    o_ref[...] = deg[:, None] * l1_ref[...] + ag + l2_ref[...]


def _combine(l1, l2, W, aggr, deg):
    N, D = l1.shape
    BN = 2048
    nblk = (N + BN - 1) // BN
    return pl.pallas_call(
        _combine_body,
        grid=(nblk,),
        in_specs=[
            pl.BlockSpec((BN, D), lambda i: (i, 0)),
            pl.BlockSpec((BN, D), lambda i: (i, 0)),
            pl.BlockSpec((D, D), lambda i: (0, 0)),
            pl.BlockSpec((NC, BN, D), lambda i: (0, i, 0)),
            pl.BlockSpec((NW, BN), lambda i: (0, i)),
        ],
        out_specs=pl.BlockSpec((BN, D), lambda i: (i, 0)),
        out_shape=jax.ShapeDtypeStruct((N, D), jnp.float32),
    )(l1, l2, W, aggr, deg)


@jax.jit
def kernel(x, edge_index, W, W1, b1, W2, b2):
    N, D = x.shape
    E = edge_index.shape[1]

    trash = N
    rpt = ((N + 1 + NS - 1) // NS + 7) // 8 * 8
    n_pad = rpt * NS

    nchunks = (E + CHUNK - 1) // CHUNK
    padn = nchunks * CHUNK - E
    ei = edge_index
    if padn:
        i = jnp.arange(padn, dtype=ei.dtype)
        pad = jnp.stack([N + 1 + i % (n_pad - N - 1), i % N])
        ei = jnp.concatenate([ei, pad], axis=1)
    nq, nr = divmod(nchunks, NW)

    l1, l2 = _lin(x, W1, b1.reshape(1, D), W2, b2.reshape(1, D))
    aggr, deg = _sc_edge_aggregate(x, ei, n_pad, nq, nr, trash)
    return _combine(l1, l2, W, aggr, deg)


# speedup vs baseline: 1.0148x; 1.0148x over previous
import functools

import jax
import jax.numpy as jnp
from jax import lax
from jax.experimental import pallas as pl
from jax.experimental.pallas import tpu as pltpu
from jax.experimental.pallas import tpu_sc as plsc

NC = 2
NS = 16
NW = NC * NS
L = 16
CHUNK = 128


def _sc_edge_aggregate(x, ei, n_pad, nq, nr, trash):
    N, D = x.shape
    rpt = n_pad // NS

    mesh = plsc.VectorSubcoreMesh(
        core_axis_name="c", subcore_axis_name="s",
        num_cores=NC, num_subcores=NS)

    def body(x_hbm, ei_hbm, aggr_hbm, deg_hbm,
             srcd0, srcd1, dstd0, dstd1, idxb0, idxb1, rb0, rb1,
             deg_v, aggr_sp, sem_i0, sem_i1, sem_g0, sem_g1, sem_s0, sem_s1):
        cid = lax.axis_index("c")
        sid = lax.axis_index("s")
        wid = cid * NS + sid
        nch = nq + jnp.where(wid < nr, 1, 0)
        base = (wid * nq + jnp.minimum(wid, nr)) * CHUNK

        zeros16 = jnp.zeros((L,), jnp.float32)
        ones16 = jnp.ones((L,), jnp.float32)
        trash16 = jnp.full((L,), trash, jnp.int32)

        def idx_dma_start(k, srcd_b, dstd_b, sem):
            off = base + k * CHUNK
            pltpu.async_copy(ei_hbm.at[0, pl.ds(off, CHUNK)], srcd_b, sem)
            pltpu.async_copy(ei_hbm.at[1, pl.ds(off, CHUNK)], dstd_b, sem)

        def idx_dma_wait(srcd_b, dstd_b, sem):
            pltpu.make_async_copy(ei_hbm.at[0, pl.ds(0, CHUNK)], srcd_b, sem).wait()
            pltpu.make_async_copy(ei_hbm.at[1, pl.ds(0, CHUNK)], dstd_b, sem).wait()

        idx_dma_start(0, srcd0, dstd0, sem_i0)
        idx_dma_start(1, srcd1, dstd1, sem_i1)

        idx_dma_wait(srcd0, dstd0, sem_i0)
        pltpu.async_copy(x_hbm.at[dstd0], rb0, sem_g0)

        def zrb(i, c):
            for cc in range(D // L):
                rb1[i, pl.ds(cc * L, L)] = zeros16
            return c
        lax.fori_loop(0, CHUNK, zrb, 0)

        def zdeg(i, c):
            deg_v[pl.ds(i * L, L)] = zeros16
            return c
        lax.fori_loop(0, n_pad // L, zdeg, 0)

        rowbase = sid * rpt
        nfull, rem = rpt // CHUNK, rpt % CHUNK
        for kk in range(nfull):
            pltpu.sync_copy(rb1, aggr_sp.at[pl.ds(rowbase + kk * CHUNK, CHUNK)])
        if rem:
            pltpu.sync_copy(rb1.at[pl.ds(0, rem)],
                            aggr_sp.at[pl.ds(rowbase + nfull * CHUNK, rem)])
        plsc.subcore_barrier()

        def step(j, srcd_b, dstd_b, idxb_b, rb_b, sem_g_b, sem_s_b,
                 srcd_o, dstd_o, idxb_o, rb_o, sem_g_o, sem_s_o,
                 sem_i_b, sem_i_o):
            for i in range(CHUNK // L):
                s16 = srcd_b[pl.ds(i * L, L)]
                d16 = dstd_b[pl.ds(i * L, L)]
                keep = s16 != d16
                idx16 = jnp.where(keep, s16, trash16)
                idxb_b[pl.ds(i * L, L)] = idx16
                plsc.addupdate_scatter(deg_v, [idx16], ones16)
            pltpu.make_async_copy(x_hbm.at[dstd_b], rb_b, sem_g_b).wait()

            @pl.when(j + 2 < nch)
            def _():
                idx_dma_start(j + 2, srcd_b, dstd_b, sem_i_b)

            @pl.when(j > 0)
            def _():
                pltpu.make_async_copy(rb_o, aggr_sp.at[idxb_o], sem_s_o).wait()

            @pl.when(j + 1 < nch)
            def _():
                idx_dma_wait(srcd_o, dstd_o, sem_i_o)
                pltpu.async_copy(x_hbm.at[dstd_o], rb_o, sem_g_o)

            pltpu.async_copy(rb_b, aggr_sp.at[idxb_b], sem_s_b, add=True)

        def loop(j, c):
            @pl.when(j % 2 == 0)
            def _():
                step(j, srcd0, dstd0, idxb0, rb0, sem_g0, sem_s0,
                     srcd1, dstd1, idxb1, rb1, sem_g1, sem_s1, sem_i0, sem_i1)

            @pl.when(j % 2 == 1)
            def _():
                step(j, srcd1, dstd1, idxb1, rb1, sem_g1, sem_s1,
                     srcd0, dstd0, idxb0, rb0, sem_g0, sem_s0, sem_i1, sem_i0)
            return c
        lax.fori_loop(0, nch, loop, 0)

        @pl.when((nch - 1) % 2 == 0)
        def _():
            pltpu.make_async_copy(rb0, aggr_sp.at[idxb0], sem_s0).wait()

        @pl.when((nch - 1) % 2 == 1)
        def _():
            pltpu.make_async_copy(rb1, aggr_sp.at[idxb1], sem_s1).wait()
        plsc.subcore_barrier()
        pltpu.sync_copy(aggr_sp.at[pl.ds(rowbase, rpt)],
                        aggr_hbm.at[cid, pl.ds(rowbase, rpt)])
        pltpu.sync_copy(deg_v, deg_hbm.at[wid])

    f = pl.kernel(
        body,
        out_type=[jax.ShapeDtypeStruct((NC, n_pad, D), jnp.float32),
                  jax.ShapeDtypeStruct((NW, n_pad), jnp.float32)],
        mesh=mesh,
        scratch_types=[
            pltpu.VMEM((CHUNK,), jnp.int32),
            pltpu.VMEM((CHUNK,), jnp.int32),
            pltpu.VMEM((CHUNK,), jnp.int32),
            pltpu.VMEM((CHUNK,), jnp.int32),
            pltpu.VMEM((CHUNK,), jnp.int32),
            pltpu.VMEM((CHUNK,), jnp.int32),
            pltpu.VMEM((CHUNK, D), jnp.float32),
            pltpu.VMEM((CHUNK, D), jnp.float32),
            pltpu.VMEM((n_pad,), jnp.float32),
            pltpu.VMEM_SHARED((n_pad, D), jnp.float32),
            pltpu.SemaphoreType.DMA,
            pltpu.SemaphoreType.DMA,
            pltpu.SemaphoreType.DMA,
            pltpu.SemaphoreType.DMA,
            pltpu.SemaphoreType.DMA,
            pltpu.SemaphoreType.DMA,
        ],
        compiler_params=pltpu.CompilerParams(needs_layout_passes=False),
    )
    return f(x, ei)


def _combine_body(x_ref, w_ref, w1_ref, b1_ref, w2_ref, b2_ref,
                  ag_ref, dg_ref, o_ref):
    x = x_ref[...]
    lin1 = jnp.dot(x, w1_ref[...], preferred_element_type=jnp.float32) + b1_ref[...]
    lin2 = jnp.dot(x, w2_ref[...], preferred_element_type=jnp.float32) + b2_ref[...]
    ax = ag_ref[0] + ag_ref[1]
    ag = jnp.dot(ax, w_ref[...], preferred_element_type=jnp.float32)
    deg = jnp.sum(dg_ref[...], axis=0)
    o_ref[...] = deg[:, None] * lin1 + ag + lin2


def _combine(x, W, W1, b1, W2, b2, aggr, deg):
    N, D = x.shape
    BN = 2048
    nblk = (N + BN - 1) // BN
    return pl.pallas_call(
        _combine_body,
        grid=(nblk,),
        in_specs=[
            pl.BlockSpec((BN, D), lambda i: (i, 0)),
            pl.BlockSpec((D, D), lambda i: (0, 0)),
            pl.BlockSpec((D, D), lambda i: (0, 0)),
            pl.BlockSpec((1, D), lambda i: (0, 0)),
            pl.BlockSpec((D, D), lambda i: (0, 0)),
            pl.BlockSpec((1, D), lambda i: (0, 0)),
            pl.BlockSpec((NC, BN, D), lambda i: (0, i, 0)),
            pl.BlockSpec((NW, BN), lambda i: (0, i)),
        ],
        out_specs=pl.BlockSpec((BN, D), lambda i: (i, 0)),
        out_shape=jax.ShapeDtypeStruct((N, D), jnp.float32),
    )(x, W, W1, b1, W2, b2, aggr, deg)


@jax.jit
def kernel(x, edge_index, W, W1, b1, W2, b2):
    N, D = x.shape
    E = edge_index.shape[1]

    trash = N
    rpt = ((N + 1 + NS - 1) // NS + 7) // 8 * 8
    n_pad = rpt * NS

    nchunks = (E + CHUNK - 1) // CHUNK
    padn = nchunks * CHUNK - E
    ei = edge_index
    if padn:
        i = jnp.arange(padn, dtype=ei.dtype)
        pad = jnp.stack([N + 1 + i % (n_pad - N - 1), i % N])
        ei = jnp.concatenate([ei, pad], axis=1)
    nq, nr = divmod(nchunks, NW)

    aggr, deg = _sc_edge_aggregate(x, ei, n_pad, nq, nr, trash)
    return _combine(x, W, W1, b1.reshape(1, D),
                    W2, b2.reshape(1, D), aggr, deg)
